# initial kernel scaffold (unmeasured)
import jax
import jax.numpy as jnp
from jax import lax
from jax.experimental import pallas as pl
from jax.experimental.pallas import tpu as pltpu

N_DEV = 32
GROUP = 16
B = 2
SQ = 128
HQ = 4
DH = 64
DMODEL = 512
DQK = HQ * DH


def kernel(x, Wq, K_ext, V_ext, Wo):
    def body(x_ref, wq_ref, k_ref, v_ref, wo_ref, out_ref,
             gkv, send_sems, recv_sems):
        my = lax.axis_index("i")
        left = (my - 2) % N_DEV
        right = (my + 2) % N_DEV

        barrier_sem = pltpu.get_barrier_semaphore()
        for nbr in (left, right):
            pl.semaphore_signal(
                barrier_sem, inc=1,
                device_id=(nbr,), device_id_type=pl.DeviceIdType.MESH,
            )
        pl.semaphore_wait(barrier_sem, 2)

        gkv[0, 0] = k_ref[...].astype(jnp.bfloat16)
        gkv[0, 1] = v_ref[...].astype(jnp.bfloat16)

        for h in range(GROUP - 1):
            rdma = pltpu.make_async_remote_copy(
                src_ref=gkv.at[h],
                dst_ref=gkv.at[h + 1],
                send_sem=send_sems.at[h],
                recv_sem=recv_sems.at[h],
                device_id=(right,),
                device_id_type=pl.DeviceIdType.MESH,
            )
            rdma.start()
            rdma.wait()

        xb = x_ref[...].reshape(B * SQ, DMODEL).astype(jnp.bfloat16)
        wq = wq_ref[...].astype(jnp.bfloat16)
        q = lax.dot(xb, wq, preferred_element_type=jnp.float32)
        q = q.astype(jnp.bfloat16)

        kv = gkv[...]

        nk = GROUP * B * SQ
        qi = lax.broadcasted_iota(jnp.int32, (B * SQ, 1), 0)
        ki = lax.broadcasted_iota(jnp.int32, (1, nk), 1)
        mask = (qi // 64) == ((ki % (B * SQ)) // 64)

        ctxs = []
        for h in range(HQ):
            kh = kv[:, 0, :, :, h, :].reshape(nk, DH)
            vh = kv[:, 1, :, :, h, :].reshape(nk, DH)
            qh = q[:, h * DH:(h + 1) * DH]
            s = lax.dot_general(
                qh, kh, (((1,), (1,)), ((), ())),
                preferred_element_type=jnp.float32,
            ) * 0.125
            s = jnp.where(mask, s, -1e9)
            m = jnp.max(s, axis=1, keepdims=True)
            e = jnp.exp(s - m)
            denom = jnp.sum(e, axis=1, keepdims=True)
            num = lax.dot(e.astype(jnp.bfloat16), vh,
                          preferred_element_type=jnp.float32)
            ctxs.append(num / denom)
        ctx = jnp.concatenate(ctxs, axis=1).astype(jnp.bfloat16)
        out = lax.dot(ctx, wo_ref[...].astype(jnp.bfloat16),
                      preferred_element_type=jnp.float32)
        out_ref[...] = out.reshape(B, SQ, DMODEL)

    return pl.pallas_call(
        body,
        out_shape=jax.ShapeDtypeStruct((B, SQ, DMODEL), jnp.float32),
        in_specs=[pl.BlockSpec(memory_space=pltpu.VMEM)] * 5,
        out_specs=pl.BlockSpec(memory_space=pltpu.VMEM),
        scratch_shapes=[
            pltpu.VMEM((GROUP, 2, B, SQ, HQ, DH), jnp.bfloat16),
            pltpu.SemaphoreType.DMA((GROUP - 1,)),
            pltpu.SemaphoreType.DMA((GROUP - 1,)),
        ],
        compiler_params=pltpu.CompilerParams(collective_id=0),
    )(x, Wq, K_ext, V_ext, Wo)


# baseline (device time: 150030 ns/iter reference)
import jax
import jax.numpy as jnp
from jax import lax
from jax.experimental import pallas as pl
from jax.experimental.pallas import tpu as pltpu

N_DEV = 32
GROUP = 16
B = 2
SQ = 128
HQ = 4
DH = 64
DMODEL = 512
DQK = HQ * DH


def kernel(x, Wq, K_ext, V_ext, Wo):
    def body(x_ref, wq_ref, k_ref, v_ref, wo_ref, out_ref,
             gkv, send_sems, recv_sems):
        my = lax.axis_index("i")
        left = (my - 2) % N_DEV
        right = (my + 2) % N_DEV

        barrier_sem = pltpu.get_barrier_semaphore()
        for nbr in (left, right):
            pl.semaphore_signal(
                barrier_sem, inc=1,
                device_id=(nbr,), device_id_type=pl.DeviceIdType.MESH,
            )
        pl.semaphore_wait(barrier_sem, 2)

        gkv[0, 0] = k_ref[...].astype(jnp.bfloat16)
        gkv[0, 1] = v_ref[...].astype(jnp.bfloat16)

        for h in range(GROUP - 1):
            rdma = pltpu.make_async_remote_copy(
                src_ref=gkv.at[h],
                dst_ref=gkv.at[h + 1],
                send_sem=send_sems.at[h],
                recv_sem=recv_sems.at[h],
                device_id=(right,),
                device_id_type=pl.DeviceIdType.MESH,
            )
            rdma.start()
            rdma.wait()

        xb = x_ref[...].reshape(B * SQ, DMODEL).astype(jnp.bfloat16)
        wq = wq_ref[...].astype(jnp.bfloat16)
        q = lax.dot(xb, wq, preferred_element_type=jnp.float32)
        q = q.astype(jnp.bfloat16)

        nk = GROUP * 64
        row_blocks = []
        for b in range(B):
            for half in range(2):
                qrow = b * SQ + half * 64
                krow = half * 64
                heads = []
                for h in range(HQ):
                    qg = q[qrow:qrow + 64, h * DH:(h + 1) * DH]
                    kh = gkv[:, 0, b, krow:krow + 64, h, :].reshape(nk, DH)
                    vh = gkv[:, 1, b, krow:krow + 64, h, :].reshape(nk, DH)
                    s = lax.dot_general(
                        qg, kh, (((1,), (1,)), ((), ())),
                        preferred_element_type=jnp.float32,
                    ) * 0.125
                    m = jnp.max(s, axis=1, keepdims=True)
                    e = jnp.exp(s - m)
                    denom = jnp.sum(e, axis=1, keepdims=True)
                    num = lax.dot(e.astype(jnp.bfloat16), vh,
                                  preferred_element_type=jnp.float32)
                    heads.append(num / denom)
                row_blocks.append(jnp.concatenate(heads, axis=1))
        ctx = jnp.concatenate(row_blocks, axis=0).astype(jnp.bfloat16)
        out = lax.dot(ctx, wo_ref[...].astype(jnp.bfloat16),
                      preferred_element_type=jnp.float32)
        out_ref[...] = out.reshape(B, SQ, DMODEL)

    return pl.pallas_call(
        body,
        out_shape=jax.ShapeDtypeStruct((B, SQ, DMODEL), jnp.float32),
        in_specs=[pl.BlockSpec(memory_space=pltpu.VMEM)] * 5,
        out_specs=pl.BlockSpec(memory_space=pltpu.VMEM),
        scratch_shapes=[
            pltpu.VMEM((GROUP, 2, B, SQ, HQ, DH), jnp.bfloat16),
            pltpu.SemaphoreType.DMA((GROUP - 1,)),
            pltpu.SemaphoreType.DMA((GROUP - 1,)),
        ],
        compiler_params=pltpu.CompilerParams(collective_id=0),
    )(x, Wq, K_ext, V_ext, Wo)
